# baseline (speedup 1.0000x reference)
import jax
import jax.numpy as jnp
from jax import lax
from jax.experimental import pallas as pl
from jax.experimental.pallas import tpu as pltpu
from jax.experimental.pallas import tpu_sc as plsc

_NUM_CORES = 2
_NUM_SUBCORES = 16
_NUM_WORKERS = _NUM_CORES * _NUM_SUBCORES
_CHUNK = 200


def kernel(x, weight):
    b, h = x.shape
    n = b * h
    dim = weight.shape[1]
    idx_per_w = n // _NUM_WORKERS
    n_chunks = idx_per_w // _CHUNK
    idx = x.T.reshape(n)

    mesh = plsc.VectorSubcoreMesh(core_axis_name="c", subcore_axis_name="s")

    @pl.kernel(
        out_type=jax.ShapeDtypeStruct((n, dim), weight.dtype),
        mesh=mesh,
        scratch_types=[
            pltpu.VMEM((idx_per_w,), jnp.int32),
            pltpu.VMEM((_CHUNK, dim), jnp.float32),
            pltpu.VMEM((_CHUNK, dim), jnp.float32),
            pltpu.SemaphoreType.DMA,
            pltpu.SemaphoreType.DMA,
            pltpu.SemaphoreType.DMA,
            pltpu.SemaphoreType.DMA,
        ],
    )
    def gather_kernel(
        w_hbm, i_hbm, o_hbm, idx_v, rows_v0, rows_v1, sem0, sem1, semw0, semw1
    ):
        wid = lax.axis_index("s") * _NUM_CORES + lax.axis_index("c")
        base = wid * idx_per_w
        pltpu.sync_copy(i_hbm.at[pl.ds(base, idx_per_w)], idx_v)

        def gather_start(c, buf, sem):
            pltpu.async_copy(
                w_hbm.at[idx_v.at[pl.ds(c * _CHUNK, _CHUNK)]], buf, sem
            )

        def gather_wait(c, buf, sem):
            pltpu.make_async_copy(
                w_hbm.at[idx_v.at[pl.ds(c * _CHUNK, _CHUNK)]], buf, sem
            ).wait()

        def wb_start(c, buf, sem):
            pltpu.async_copy(buf, o_hbm.at[pl.ds(base + c * _CHUNK, _CHUNK)], sem)

        def wb_drain(c, buf, sem):
            pltpu.make_async_copy(
                buf, o_hbm.at[pl.ds(base + c * _CHUNK, _CHUNK)], sem
            ).wait()

        gather_start(0, rows_v0, sem0)
        gather_start(1, rows_v1, sem1)

        @pl.loop(0, n_chunks, step=2)
        def _(c):
            gather_wait(c, rows_v0, sem0)
            wb_start(c, rows_v0, semw0)
            wb_drain(c, rows_v0, semw0)

            @pl.when(c + 2 < n_chunks)
            def _():
                gather_start(c + 2, rows_v0, sem0)

            gather_wait(c + 1, rows_v1, sem1)
            wb_start(c + 1, rows_v1, semw1)
            wb_drain(c + 1, rows_v1, semw1)

            @pl.when(c + 3 < n_chunks)
            def _():
                gather_start(c + 3, rows_v1, sem1)

    out = gather_kernel(weight, idx)
    return out.reshape(h, b, dim).transpose(1, 0, 2)


# speedup vs baseline: 1.0012x; 1.0012x over previous
import jax
import jax.numpy as jnp
from jax import lax
from jax.experimental import pallas as pl
from jax.experimental.pallas import tpu as pltpu
from jax.experimental.pallas import tpu_sc as plsc

_NUM_CORES = 2
_NUM_SUBCORES = 16
_NUM_WORKERS = _NUM_CORES * _NUM_SUBCORES


def kernel(x, weight):
    b, h = x.shape
    n = b * h
    dim = weight.shape[1]
    cols = b // _NUM_WORKERS
    idx2 = x.T

    mesh = plsc.VectorSubcoreMesh(core_axis_name="c", subcore_axis_name="s")

    @pl.kernel(
        out_type=jax.ShapeDtypeStruct((n, dim), weight.dtype),
        mesh=mesh,
        scratch_types=[
            pltpu.VMEM((h, cols), jnp.int32),
            pltpu.VMEM((cols, dim), jnp.float32),
            pltpu.VMEM((cols, dim), jnp.float32),
            pltpu.SemaphoreType.DMA,
            pltpu.SemaphoreType.DMA,
            pltpu.SemaphoreType.DMA,
            pltpu.SemaphoreType.DMA,
        ],
    )
    def gather_kernel(
        w_hbm, i_hbm, o_hbm, idx_v, rows_v0, rows_v1, sem0, sem1, semw0, semw1
    ):
        wid = lax.axis_index("s") * _NUM_CORES + lax.axis_index("c")
        col0 = wid * cols
        pltpu.sync_copy(i_hbm.at[:, pl.ds(col0, cols)], idx_v)

        def gather_start(r, buf, sem):
            pltpu.async_copy(w_hbm.at[idx_v.at[r]], buf, sem)

        def gather_wait(r, buf, sem):
            pltpu.make_async_copy(w_hbm.at[idx_v.at[r]], buf, sem).wait()

        def wb_start(r, buf, sem):
            pltpu.async_copy(buf, o_hbm.at[pl.ds(r * b + col0, cols)], sem)

        def wb_drain(r, buf, sem):
            pltpu.make_async_copy(
                buf, o_hbm.at[pl.ds(r * b + col0, cols)], sem
            ).wait()

        gather_start(0, rows_v0, sem0)
        gather_start(1, rows_v1, sem1)

        @pl.loop(0, h, step=2)
        def _(r):
            gather_wait(r, rows_v0, sem0)
            wb_start(r, rows_v0, semw0)
            wb_drain(r, rows_v0, semw0)

            @pl.when(r + 2 < h)
            def _():
                gather_start(r + 2, rows_v0, sem0)

            gather_wait(r + 1, rows_v1, sem1)
            wb_start(r + 1, rows_v1, semw1)
            wb_drain(r + 1, rows_v1, semw1)

            @pl.when(r + 3 < h)
            def _():
                gather_start(r + 3, rows_v1, sem1)

    out = gather_kernel(weight, idx2)
    return out.reshape(h, b, dim).transpose(1, 0, 2)


# speedup vs baseline: 1.0276x; 1.0265x over previous
import jax
import jax.numpy as jnp
from jax import lax
from jax.experimental import pallas as pl
from jax.experimental.pallas import tpu as pltpu
from jax.experimental.pallas import tpu_sc as plsc

_NUM_CORES = 2
_NUM_SUBCORES = 16
_NUM_WORKERS = _NUM_CORES * _NUM_SUBCORES
_DEPTH = 5


def kernel(x, weight):
    b, h = x.shape
    n = b * h
    dim = weight.shape[1]
    cols = b // _NUM_WORKERS
    idx2 = x.T

    mesh = plsc.VectorSubcoreMesh(core_axis_name="c", subcore_axis_name="s")

    @pl.kernel(
        out_type=jax.ShapeDtypeStruct((n, dim), weight.dtype),
        mesh=mesh,
        scratch_types=[pltpu.VMEM((h, cols), jnp.int32)]
        + [pltpu.VMEM((cols, dim), jnp.float32)] * _DEPTH
        + [pltpu.SemaphoreType.DMA] * (2 * _DEPTH),
    )
    def gather_kernel(w_hbm, i_hbm, o_hbm, idx_v, *bufs_sems):
        bufs = bufs_sems[:_DEPTH]
        semg = bufs_sems[_DEPTH : 2 * _DEPTH]
        semw = bufs_sems[2 * _DEPTH :]
        wid = lax.axis_index("s") * _NUM_CORES + lax.axis_index("c")
        col0 = wid * cols
        pltpu.sync_copy(i_hbm.at[:, pl.ds(col0, cols)], idx_v)

        def gather_start(r, k):
            pltpu.async_copy(w_hbm.at[idx_v.at[r]], bufs[k], semg[k])

        def gather_wait(r, k):
            pltpu.make_async_copy(w_hbm.at[idx_v.at[r]], bufs[k], semg[k]).wait()

        def wb_start(r, k):
            pltpu.async_copy(
                bufs[k], o_hbm.at[pl.ds(r * b + col0, cols)], semw[k]
            )

        def wb_drain(r, k):
            pltpu.make_async_copy(
                bufs[k], o_hbm.at[pl.ds(r * b + col0, cols)], semw[k]
            ).wait()

        for k in range(_DEPTH):
            gather_start(k, k)

        @pl.loop(0, h, step=_DEPTH)
        def _(r):
            for k in range(_DEPTH):
                gather_wait(r + k, k)
                wb_start(r + k, k)
                wb_drain(r + k, k)

                @pl.when(r + k + _DEPTH < h)
                def _(r=r, k=k):
                    gather_start(r + k + _DEPTH, k)

    out = gather_kernel(weight, idx2)
    return out.reshape(h, b, dim).transpose(1, 0, 2)


# speedup vs baseline: 1.0331x; 1.0053x over previous
import jax
import jax.numpy as jnp
from jax import lax
from jax.experimental import pallas as pl
from jax.experimental.pallas import tpu as pltpu
from jax.experimental.pallas import tpu_sc as plsc

_NUM_CORES = 2
_NUM_SUBCORES = 16
_NUM_WORKERS = _NUM_CORES * _NUM_SUBCORES
_DEPTH = 10


def kernel(x, weight):
    b, h = x.shape
    n = b * h
    dim = weight.shape[1]
    cols = b // _NUM_WORKERS
    half = cols // 2
    n_chunks = 2 * h
    idx2 = x.T

    mesh = plsc.VectorSubcoreMesh(core_axis_name="c", subcore_axis_name="s")

    @pl.kernel(
        out_type=jax.ShapeDtypeStruct((n, dim), weight.dtype),
        mesh=mesh,
        scratch_types=[pltpu.VMEM((h, cols), jnp.int32)]
        + [pltpu.VMEM((half, dim), jnp.float32)] * _DEPTH
        + [pltpu.SemaphoreType.DMA] * (2 * _DEPTH),
    )
    def gather_kernel(w_hbm, i_hbm, o_hbm, idx_v, *bufs_sems):
        bufs = bufs_sems[:_DEPTH]
        semg = bufs_sems[_DEPTH : 2 * _DEPTH]
        semw = bufs_sems[2 * _DEPTH :]
        wid = lax.axis_index("s") * _NUM_CORES + lax.axis_index("c")
        col0 = wid * cols
        pltpu.sync_copy(i_hbm.at[:, pl.ds(col0, cols)], idx_v)

        def gather_start(c, k):
            r = c // 2
            s = (c % 2) * half
            pltpu.async_copy(
                w_hbm.at[idx_v.at[r, pl.ds(s, half)]], bufs[k], semg[k]
            )

        def gather_wait(c, k):
            r = c // 2
            s = (c % 2) * half
            pltpu.make_async_copy(
                w_hbm.at[idx_v.at[r, pl.ds(s, half)]], bufs[k], semg[k]
            ).wait()

        def wb_start(c, k):
            r = c // 2
            s = (c % 2) * half
            pltpu.async_copy(
                bufs[k], o_hbm.at[pl.ds(r * b + col0 + s, half)], semw[k]
            )

        def wb_drain(c, k):
            r = c // 2
            s = (c % 2) * half
            pltpu.make_async_copy(
                bufs[k], o_hbm.at[pl.ds(r * b + col0 + s, half)], semw[k]
            ).wait()

        for k in range(_DEPTH):
            gather_start(k, k)

        @pl.loop(0, n_chunks, step=_DEPTH)
        def _(c):
            for k in range(_DEPTH):
                gather_wait(c + k, k)
                wb_start(c + k, k)
                wb_drain(c + k, k)

                @pl.when(c + k + _DEPTH < n_chunks)
                def _(c=c, k=k):
                    gather_start(c + k + _DEPTH, k)

    out = gather_kernel(weight, idx2)
    return out.reshape(h, b, dim).transpose(1, 0, 2)
